# baseline (speedup 1.0000x reference)
import functools

import jax
import jax.numpy as jnp
from jax import lax
from jax.experimental import pallas as pl
from jax.experimental.pallas import tpu as pltpu

B = 4096
N = 50
NFEAT = 128
NHID = 64
NCLASS = 16

BB = 64


def _body(su_ref, sv_ref, sua_ref, sva_ref,
          w1_ref, b1_ref, w2_ref, b2_ref,
          f1w_ref, f1b_ref, f2w_ref, f2b_ref,
          f3w_ref, f3b_ref, f4w_ref, f4b_ref,
          out_ref):
    w1 = w1_ref[...]
    b1 = b1_ref[...]
    w2 = w2_ref[...]
    b2 = b2_ref[...]

    def gcn2(x, adj):
        s1 = lax.dot_general(x, w1, (((2,), (0,)), ((), ())),
                             preferred_element_type=jnp.float32)
        h1 = lax.dot_general(adj, s1, (((2,), (1,)), ((0,), (0,))),
                             preferred_element_type=jnp.float32)
        h1 = jnp.maximum(h1 + b1[None, None, :], 0.0)
        s2 = lax.dot_general(h1, w2, (((2,), (0,)), ((), ())),
                             preferred_element_type=jnp.float32)
        g2 = lax.dot_general(adj, s2, (((2,), (1,)), ((0,), (0,))),
                             preferred_element_type=jnp.float32)
        return g2 + b2[None, None, :]

    su = gcn2(su_ref[...], sua_ref[...])
    sv = gcn2(sv_ref[...], sva_ref[...])
    data = jnp.concatenate([su, sv], axis=1)
    data = data.reshape(BB, 2 * N * NCLASS)

    d = jnp.maximum(
        jnp.dot(data, f1w_ref[...], preferred_element_type=jnp.float32)
        + f1b_ref[...][None, :], 0.0)
    d = jnp.maximum(
        jnp.dot(d, f2w_ref[...], preferred_element_type=jnp.float32)
        + f2b_ref[...][None, :], 0.0)
    d = jnp.maximum(
        jnp.dot(d, f3w_ref[...], preferred_element_type=jnp.float32)
        + f3b_ref[...][None, :], 0.0)
    d = (jnp.dot(d, f4w_ref[...], preferred_element_type=jnp.float32)
         + f4b_ref[...][None, :])
    out_ref[...] = d


@jax.jit
def kernel(solute, solvent, solute_adj, solvent_adj,
           gc1_w, gc1_b, gc2_w, gc2_b,
           fc1_w, fc1_b, fc2_w, fc2_b, fc3_w, fc3_b, fc4_w, fc4_b):
    grid = (B // BB,)

    def batch_spec(shape):
        return pl.BlockSpec((BB,) + shape,
                            lambda i: (i,) + (0,) * len(shape))

    def full_spec(arr):
        nd = arr.ndim
        return pl.BlockSpec(arr.shape, lambda i: (0,) * nd)

    in_specs = [
        batch_spec((N, NFEAT)),
        batch_spec((N, NFEAT)),
        batch_spec((N, N)),
        batch_spec((N, N)),
        full_spec(gc1_w), full_spec(gc1_b),
        full_spec(gc2_w), full_spec(gc2_b),
        full_spec(fc1_w), full_spec(fc1_b),
        full_spec(fc2_w), full_spec(fc2_b),
        full_spec(fc3_w), full_spec(fc3_b),
        full_spec(fc4_w), full_spec(fc4_b),
    ]

    out = pl.pallas_call(
        _body,
        grid=grid,
        in_specs=in_specs,
        out_specs=pl.BlockSpec((BB, 1), lambda i: (i, 0)),
        out_shape=jax.ShapeDtypeStruct((B, 1), jnp.float32),
        compiler_params=pltpu.CompilerParams(
            dimension_semantics=("arbitrary",),
        ),
    )(solute, solvent, solute_adj, solvent_adj,
      gc1_w, gc1_b, gc2_w, gc2_b,
      fc1_w, fc1_b, fc2_w, fc2_b, fc3_w, fc3_b, fc4_w, fc4_b)
    return out
